# baseline (speedup 1.0000x reference)
import jax
import jax.numpy as jnp
from jax import lax
from jax.experimental import pallas as pl
from jax.experimental.pallas import tpu as pltpu
from jax.experimental.pallas import tpu_sc as plsc

_NUM_EMBED = 384
_N = 200000
_B = 128
_NBLK_FULL = _N // _B
_NC, _NS, _L = 2, 16, 16
_NW = _NC * _NS
_T = 49


def _body(fused_hbm, px_hbm, py_hbm, pz_hbm, d_hbm, out_hbm, shared,
          px0, py0, pz0, d0, px1, py1, pz1, d1,
          ix0, iy0, iz0, ix1, iy1, iz1,
          bx0, by0, bz0, bx1, by1, bz1,
          sem_i0, sem_i1, sem_g0, sem_g1, sem_w0, sem_w1):
  cid = lax.axis_index("c")
  sid = lax.axis_index("s")
  wid = sid * _NC + cid

  pxv = (px0, px1)
  pyv = (py0, py1)
  pzv = (pz0, pz1)
  dv = (d0, d1)
  ix = (ix0, ix1)
  iy = (iy0, iy1)
  iz = (iz0, iz1)
  bx = (bx0, bx1)
  by = (by0, by1)
  bz = (bz0, bz1)
  sem_i = (sem_i0, sem_i1)
  sem_g = (sem_g0, sem_g1)
  sem_w = (sem_w0, sem_w1)

  @pl.when(sid == 0)
  def _stage():
    pltpu.sync_copy(fused_hbm, shared)

  plsc.subcore_barrier()

  def node_base(t):
    b = wid + t * _NW
    base = jnp.minimum(b, _NBLK_FULL) * _B
    base = base - jnp.where(b == _NBLK_FULL, _B // 2, 0)
    return jnp.where(b > _NBLK_FULL, 0, base)

  def idx_descs(t, s):
    base = node_base(t)
    sl = pl.ds(base, _B)
    return (
        pltpu.make_async_copy(px_hbm.at[sl], pxv[s], sem_i[s]),
        pltpu.make_async_copy(py_hbm.at[sl], pyv[s], sem_i[s]),
        pltpu.make_async_copy(pz_hbm.at[sl], pzv[s], sem_i[s]),
        pltpu.make_async_copy(d_hbm.at[sl], dv[s], sem_i[s]),
    )

  def gather_descs(s):
    return (
        pltpu.make_async_copy(shared.at[ix[s]], bx[s], sem_g[s]),
        pltpu.make_async_copy(shared.at[iy[s]], by[s], sem_g[s]),
        pltpu.make_async_copy(shared.at[iz[s]], bz[s], sem_g[s]),
    )

  def write_descs(t, s):
    base = node_base(t)
    return (
        pltpu.make_async_copy(
            bx[s], out_hbm.at[pl.ds(base, _B), pl.ds(0, 128)], sem_w[s]),
        pltpu.make_async_copy(
            by[s], out_hbm.at[pl.ds(base, _B), pl.ds(128, 128)], sem_w[s]),
        pltpu.make_async_copy(
            bz[s], out_hbm.at[pl.ds(base, _B), pl.ds(256, 128)], sem_w[s]),
    )

  def compute_indices(s):
    for g in range(_B // _L):
      sl = pl.ds(g * _L, _L)
      d = dv[s][sl]
      ix[s][sl] = pxv[s][sl] * 4 + d
      iy[s][sl] = pyv[s][sl] * 4 + d + 512
      iz[s][sl] = pzv[s][sl] * 4 + d + 1024

  def step(t, s, first=False):
    if not first:
      for c in write_descs(t - 2, s):
        c.wait()
    for c in idx_descs(t, s):
      c.wait()
    compute_indices(s)
    for c in gather_descs(s):
      c.start()
    for c in gather_descs(s):
      c.wait()
    for c in write_descs(t, s):
      c.start()
    for c in idx_descs(t + 2, s):
      c.start()

  for c in idx_descs(0, 0):
    c.start()
  for c in idx_descs(1, 1):
    c.start()
  step(0, 0, first=True)
  step(1, 1, first=True)

  def loop_body(i, carry):
    step(2 * i, 0)
    step(2 * i + 1, 1)
    return carry

  lax.fori_loop(1, (_T - 1) // 2, loop_body, 0)

  step(_T - 1, 0)
  for c in idx_descs(_T, 1):
    c.wait()
  for c in idx_descs(_T + 1, 0):
    c.wait()
  for c in write_descs(_T - 2, 1):
    c.wait()
  for c in write_descs(_T - 1, 0):
    c.wait()


@jax.jit
def _run(fused, px, py, pz, d):
  mesh = plsc.VectorSubcoreMesh(core_axis_name="c", subcore_axis_name="s")
  return pl.kernel(
      _body,
      out_type=jax.ShapeDtypeStruct((_N, _NUM_EMBED), jnp.float32),
      mesh=mesh,
      scratch_types=[
          pltpu.VMEM_SHARED((3 * 512, 128), jnp.float32),
          pltpu.VMEM((_B,), jnp.int32),
          pltpu.VMEM((_B,), jnp.int32),
          pltpu.VMEM((_B,), jnp.int32),
          pltpu.VMEM((_B,), jnp.int32),
          pltpu.VMEM((_B,), jnp.int32),
          pltpu.VMEM((_B,), jnp.int32),
          pltpu.VMEM((_B,), jnp.int32),
          pltpu.VMEM((_B,), jnp.int32),
          pltpu.VMEM((_B,), jnp.int32),
          pltpu.VMEM((_B,), jnp.int32),
          pltpu.VMEM((_B,), jnp.int32),
          pltpu.VMEM((_B,), jnp.int32),
          pltpu.VMEM((_B,), jnp.int32),
          pltpu.VMEM((_B,), jnp.int32),
          pltpu.VMEM((_B, 128), jnp.float32),
          pltpu.VMEM((_B, 128), jnp.float32),
          pltpu.VMEM((_B, 128), jnp.float32),
          pltpu.VMEM((_B, 128), jnp.float32),
          pltpu.VMEM((_B, 128), jnp.float32),
          pltpu.VMEM((_B, 128), jnp.float32),
          pltpu.SemaphoreType.DMA,
          pltpu.SemaphoreType.DMA,
          pltpu.SemaphoreType.DMA,
          pltpu.SemaphoreType.DMA,
          pltpu.SemaphoreType.DMA,
          pltpu.SemaphoreType.DMA,
      ],
  )(fused, px, py, pz, d)


def kernel(data, xyz, depth_idx, absolute_emb, depth_table):
  del data
  tabs = [absolute_emb[:, a::3] for a in range(3)]
  dchunks = [depth_table[:, 128 * a:128 * (a + 1)] for a in range(3)]
  fused = jnp.concatenate(
      [(t[:, None, :] + dc[None, :, :]).reshape(512, 128)
       for t, dc in zip(tabs, dchunks)], axis=0)

  return _run(fused, xyz[:, 0], xyz[:, 1], xyz[:, 2], depth_idx)


# speedup vs baseline: 1.0107x; 1.0107x over previous
import functools
import jax
import jax.numpy as jnp
import numpy as np
from jax import lax
from jax.experimental import pallas as pl
from jax.experimental.pallas import tpu as pltpu
from jax.experimental.pallas import tpu_sc as plsc

_NUM_EMBED = 384
_N = 200000
_B = 128
_NBLK_FULL = _N // _B
_NC, _NS, _L = 2, 16, 16
_NW = _NC * _NS
_T = 49
_NBLK = _NW * _T


def _body(fused_hbm, idx_hbm, out_hbm, shared, idxw,
          ix0, iy0, iz0, ix1, iy1, iz1,
          bx0, by0, bz0, bx1, by1, bz1,
          sem_i, sem_g0, sem_g1, sem_w0, sem_w1):
  cid = lax.axis_index("c")
  sid = lax.axis_index("s")
  wid = sid * _NC + cid

  ix = (ix0, ix1)
  iy = (iy0, iy1)
  iz = (iz0, iz1)
  bx = (bx0, bx1)
  by = (by0, by1)
  bz = (bz0, bz1)
  sem_g = (sem_g0, sem_g1)
  sem_w = (sem_w0, sem_w1)

  ci = pltpu.async_copy(idx_hbm.at[wid], idxw, sem_i)

  @pl.when(sid == 0)
  def _stage():
    pltpu.sync_copy(fused_hbm, shared)

  plsc.subcore_barrier()
  ci.wait()

  def out_base(t):
    b = wid * _T + t
    base = jnp.minimum(b, _NBLK_FULL) * _B
    base = base - jnp.where(b == _NBLK_FULL, _B // 2, 0)
    return jnp.where(b > _NBLK_FULL, 0, base)

  def start_block(t, s):
    for g in range(_B // _L):
      sl = pl.ds(g * _L, _L)
      w = idxw[t, sl]
      d = lax.shift_right_logical(w, 24)
      px = w & 0xFF
      py = lax.shift_right_logical(w, 8) & 0xFF
      pz = lax.shift_right_logical(w, 16) & 0xFF
      ix[s][sl] = px * 4 + d
      iy[s][sl] = py * 4 + d + 512
      iz[s][sl] = pz * 4 + d + 1024
    pltpu.make_async_copy(shared.at[ix[s]], bx[s], sem_g[s]).start()
    pltpu.make_async_copy(shared.at[iy[s]], by[s], sem_g[s]).start()
    pltpu.make_async_copy(shared.at[iz[s]], bz[s], sem_g[s]).start()

  def wait_gathers(s):
    pltpu.make_async_copy(shared.at[ix[s]], bx[s], sem_g[s]).wait()
    pltpu.make_async_copy(shared.at[iy[s]], by[s], sem_g[s]).wait()
    pltpu.make_async_copy(shared.at[iz[s]], bz[s], sem_g[s]).wait()

  def write_descs(t, s):
    base = out_base(t)
    return (
        pltpu.make_async_copy(
            bx[s], out_hbm.at[pl.ds(base, _B), pl.ds(0, 128)], sem_w[s]),
        pltpu.make_async_copy(
            by[s], out_hbm.at[pl.ds(base, _B), pl.ds(128, 128)], sem_w[s]),
        pltpu.make_async_copy(
            bz[s], out_hbm.at[pl.ds(base, _B), pl.ds(256, 128)], sem_w[s]),
    )

  def issue_writes(t, s):
    for c in write_descs(t, s):
      c.start()

  def wait_writes(t, s):
    for c in write_descs(t, s):
      c.wait()

  start_block(0, 0)
  start_block(1, 1)
  wait_gathers(0)
  issue_writes(0, 0)
  wait_gathers(1)
  issue_writes(1, 1)

  def loop_body(i, carry):
    t0 = 2 * i
    wait_writes(t0 - 2, 0)
    start_block(t0, 0)
    wait_gathers(0)
    issue_writes(t0, 0)
    wait_writes(t0 - 1, 1)
    start_block(t0 + 1, 1)
    wait_gathers(1)
    issue_writes(t0 + 1, 1)
    return carry

  lax.fori_loop(1, (_T - 1) // 2, loop_body, 0)

  t_last = _T - 1
  wait_writes(t_last - 2, 0)
  start_block(t_last, 0)
  wait_gathers(0)
  issue_writes(t_last, 0)
  wait_writes(t_last - 1, 1)
  wait_writes(t_last, 0)


@jax.jit
def _run(fused, idx_packed):
  mesh = plsc.VectorSubcoreMesh(core_axis_name="c", subcore_axis_name="s")
  return pl.kernel(
      _body,
      out_type=jax.ShapeDtypeStruct((_N, _NUM_EMBED), jnp.float32),
      mesh=mesh,
      scratch_types=[
          pltpu.VMEM_SHARED((3 * 512, 128), jnp.float32),
          pltpu.VMEM((_T, _B), jnp.int32),
          pltpu.VMEM((_B,), jnp.int32),
          pltpu.VMEM((_B,), jnp.int32),
          pltpu.VMEM((_B,), jnp.int32),
          pltpu.VMEM((_B,), jnp.int32),
          pltpu.VMEM((_B,), jnp.int32),
          pltpu.VMEM((_B,), jnp.int32),
          pltpu.VMEM((_B, 128), jnp.float32),
          pltpu.VMEM((_B, 128), jnp.float32),
          pltpu.VMEM((_B, 128), jnp.float32),
          pltpu.VMEM((_B, 128), jnp.float32),
          pltpu.VMEM((_B, 128), jnp.float32),
          pltpu.VMEM((_B, 128), jnp.float32),
          pltpu.SemaphoreType.DMA,
          pltpu.SemaphoreType.DMA,
          pltpu.SemaphoreType.DMA,
          pltpu.SemaphoreType.DMA,
          pltpu.SemaphoreType.DMA,
      ],
  )(fused, idx_packed)


def kernel(data, xyz, depth_idx, absolute_emb, depth_table):
  del data
  tabs = [absolute_emb[:, a::3] for a in range(3)]
  dchunks = [depth_table[:, 128 * a:128 * (a + 1)] for a in range(3)]
  fused = jnp.concatenate(
      [(t[:, None, :] + dc[None, :, :]).reshape(512, 128)
       for t, dc in zip(tabs, dchunks)], axis=0)

  idxs = (xyz[:, 0] | (xyz[:, 1] << 8) | (xyz[:, 2] << 16)
          | (depth_idx << 24))
  main = idxs[:_NBLK_FULL * _B]
  tail = idxs[_N - _B:]
  n_fill = _NBLK - _NBLK_FULL - 1
  fill = jnp.tile(idxs[:_B], (n_fill,))
  idx_packed = jnp.concatenate([main, tail, fill]).reshape(_NW, _T, _B)

  return _run(fused, idx_packed)


# speedup vs baseline: 1.0207x; 1.0099x over previous
import functools
import jax
import jax.numpy as jnp
import numpy as np
from jax import lax
from jax.experimental import pallas as pl
from jax.experimental.pallas import tpu as pltpu
from jax.experimental.pallas import tpu_sc as plsc

_NUM_EMBED = 384
_N = 200000
_B = 128
_NBLK_FULL = _N // _B
_NC, _NS, _L = 2, 16, 16
_NW = _NC * _NS
_T = 49
_NBLK = _NW * _T


def _body(fused_hbm, idx_hbm, out_hbm, shared, idxw,
          ix0, iy0, iz0, ix1, iy1, iz1,
          bx0, by0, bz0, bx1, by1, bz1,
          sem_i, sem_gx0, sem_gy0, sem_gz0, sem_gx1, sem_gy1, sem_gz1,
          sem_w0, sem_w1):
  cid = lax.axis_index("c")
  sid = lax.axis_index("s")
  wid = sid * _NC + cid

  ix = (ix0, ix1)
  iy = (iy0, iy1)
  iz = (iz0, iz1)
  bx = (bx0, bx1)
  by = (by0, by1)
  bz = (bz0, bz1)
  sem_gx = (sem_gx0, sem_gx1)
  sem_gy = (sem_gy0, sem_gy1)
  sem_gz = (sem_gz0, sem_gz1)
  sem_w = (sem_w0, sem_w1)

  ci = pltpu.async_copy(idx_hbm.at[wid], idxw, sem_i)

  @pl.when(sid == 0)
  def _stage():
    pltpu.sync_copy(fused_hbm, shared)

  plsc.subcore_barrier()
  ci.wait()

  def out_base(t):
    b = wid * _T + t
    base = jnp.minimum(b, _NBLK_FULL) * _B
    base = base - jnp.where(b == _NBLK_FULL, _B // 2, 0)
    return jnp.where(b > _NBLK_FULL, 0, base)

  def start_block(t, s):
    for g in range(_B // _L):
      sl = pl.ds(g * _L, _L)
      w = idxw[t, sl]
      d = lax.shift_right_logical(w, 24)
      px = w & 0xFF
      py = lax.shift_right_logical(w, 8) & 0xFF
      pz = lax.shift_right_logical(w, 16) & 0xFF
      ix[s][sl] = px * 4 + d
      iy[s][sl] = py * 4 + d + 512
      iz[s][sl] = pz * 4 + d + 1024
    pltpu.make_async_copy(shared.at[ix[s]], bx[s], sem_gx[s]).start()
    pltpu.make_async_copy(shared.at[iy[s]], by[s], sem_gy[s]).start()
    pltpu.make_async_copy(shared.at[iz[s]], bz[s], sem_gz[s]).start()

  def wait_gather_x(s):
    pltpu.make_async_copy(shared.at[ix[s]], bx[s], sem_gx[s]).wait()

  def wait_gather_y(s):
    pltpu.make_async_copy(shared.at[iy[s]], by[s], sem_gy[s]).wait()

  def wait_gather_z(s):
    pltpu.make_async_copy(shared.at[iz[s]], bz[s], sem_gz[s]).wait()

  def write_descs(t, s):
    base = out_base(t)
    return (
        pltpu.make_async_copy(
            bx[s], out_hbm.at[pl.ds(base, _B), pl.ds(0, 128)], sem_w[s]),
        pltpu.make_async_copy(
            by[s], out_hbm.at[pl.ds(base, _B), pl.ds(128, 128)], sem_w[s]),
        pltpu.make_async_copy(
            bz[s], out_hbm.at[pl.ds(base, _B), pl.ds(256, 128)], sem_w[s]),
    )

  def finish_block(t, s):
    wx, wy, wz = write_descs(t, s)
    wait_gather_x(s)
    wx.start()
    wait_gather_y(s)
    wy.start()
    wait_gather_z(s)
    wz.start()

  def wait_writes(t, s):
    for c in write_descs(t, s):
      c.wait()

  start_block(0, 0)
  start_block(1, 1)
  finish_block(0, 0)
  finish_block(1, 1)

  def loop_body(i, carry):
    t0 = 2 * i
    wait_writes(t0 - 2, 0)
    start_block(t0, 0)
    finish_block(t0, 0)
    wait_writes(t0 - 1, 1)
    start_block(t0 + 1, 1)
    finish_block(t0 + 1, 1)
    return carry

  lax.fori_loop(1, (_T - 1) // 2, loop_body, 0)

  t_last = _T - 1
  wait_writes(t_last - 2, 0)
  start_block(t_last, 0)
  finish_block(t_last, 0)
  wait_writes(t_last - 1, 1)
  wait_writes(t_last, 0)


@jax.jit
def _run(fused, idx_packed):
  mesh = plsc.VectorSubcoreMesh(core_axis_name="c", subcore_axis_name="s")
  return pl.kernel(
      _body,
      out_type=jax.ShapeDtypeStruct((_N, _NUM_EMBED), jnp.float32),
      mesh=mesh,
      scratch_types=[
          pltpu.VMEM_SHARED((3 * 512, 128), jnp.float32),
          pltpu.VMEM((_T, _B), jnp.int32),
          pltpu.VMEM((_B,), jnp.int32),
          pltpu.VMEM((_B,), jnp.int32),
          pltpu.VMEM((_B,), jnp.int32),
          pltpu.VMEM((_B,), jnp.int32),
          pltpu.VMEM((_B,), jnp.int32),
          pltpu.VMEM((_B,), jnp.int32),
          pltpu.VMEM((_B, 128), jnp.float32),
          pltpu.VMEM((_B, 128), jnp.float32),
          pltpu.VMEM((_B, 128), jnp.float32),
          pltpu.VMEM((_B, 128), jnp.float32),
          pltpu.VMEM((_B, 128), jnp.float32),
          pltpu.VMEM((_B, 128), jnp.float32),
          pltpu.SemaphoreType.DMA,
          pltpu.SemaphoreType.DMA,
          pltpu.SemaphoreType.DMA,
          pltpu.SemaphoreType.DMA,
          pltpu.SemaphoreType.DMA,
          pltpu.SemaphoreType.DMA,
          pltpu.SemaphoreType.DMA,
          pltpu.SemaphoreType.DMA,
          pltpu.SemaphoreType.DMA,
      ],
  )(fused, idx_packed)


def kernel(data, xyz, depth_idx, absolute_emb, depth_table):
  del data
  tabs = [absolute_emb[:, a::3] for a in range(3)]
  dchunks = [depth_table[:, 128 * a:128 * (a + 1)] for a in range(3)]
  fused = jnp.concatenate(
      [(t[:, None, :] + dc[None, :, :]).reshape(512, 128)
       for t, dc in zip(tabs, dchunks)], axis=0)

  idxs = (xyz[:, 0] | (xyz[:, 1] << 8) | (xyz[:, 2] << 16)
          | (depth_idx << 24))
  main = idxs[:_NBLK_FULL * _B]
  tail = idxs[_N - _B:]
  n_fill = _NBLK - _NBLK_FULL - 1
  fill = jnp.tile(idxs[:_B], (n_fill,))
  idx_packed = jnp.concatenate([main, tail, fill]).reshape(_NW, _T, _B)

  return _run(fused, idx_packed)
